# baseline (speedup 1.0000x reference)
import functools

import jax
import jax.numpy as jnp
from jax import lax
from jax.experimental import pallas as pl
from jax.experimental.pallas import tpu as pltpu
from jax.experimental.pallas import tpu_sc as plsc

B = 4096
V = 100000
D = 64
L = 20
NC = 2
NS = 16
NW = NC * NS
BPW = B // NW
LANES = 16
V2 = V // 2
PW = 2 * D
NBUF = 4


def _sc_pooled_lookup(user_indices, midx_t, ut2, mt2):
  mesh = plsc.VectorSubcoreMesh(
      core_axis_name="c", subcore_axis_name="s",
      num_cores=NC, num_subcores=NS)

  @functools.partial(
      pl.kernel,
      out_type=(jax.ShapeDtypeStruct((D, B), jnp.float32),
                jax.ShapeDtypeStruct((D, B), jnp.float32)),
      mesh=mesh,
      compiler_params=pltpu.CompilerParams(
          use_tc_tiling_on_sc=True, needs_layout_passes=False),
      scratch_types=[
          pltpu.VMEM((BPW,), jnp.int32),
          pltpu.VMEM((BPW,), jnp.int32),
          pltpu.VMEM((L, BPW), jnp.int32),
          pltpu.VMEM((L, BPW), jnp.int32),
          pltpu.VMEM((L, BPW), jnp.int32),
          pltpu.VMEM((BPW, PW), jnp.float32),
          pltpu.VMEM((NBUF, BPW, PW), jnp.float32),
          pltpu.VMEM((D, BPW), jnp.float32),
          pltpu.VMEM((D, BPW), jnp.float32),
          pltpu.SemaphoreType.DMA,
          [pltpu.SemaphoreType.DMA] * NBUF,
      ],
  )
  def k(uidx_hbm, midx_hbm, ut_hbm, mt_hbm, ue_out, me_out,
        urow_v, ucol_v, midx_v, mrow_v, mcol_v,
        upairs, mring, usel, pooled, usem, msems):
    wid = lax.axis_index("s") * NC + lax.axis_index("c")
    base = wid * BPW

    pltpu.sync_copy(uidx_hbm.at[pl.ds(base, BPW)], urow_v)
    pltpu.sync_copy(midx_hbm.at[:, pl.ds(base, BPW)], midx_v)

    @plsc.parallel_loop(0, BPW // LANES)
    def _(j0):
      v = urow_v[pl.ds(j0 * LANES, LANES)]
      ucol_v[pl.ds(j0 * LANES, LANES)] = (v & 1) << 6
      urow_v[pl.ds(j0 * LANES, LANES)] = v >> 1

    ucopy = pltpu.async_copy(ut_hbm.at[urow_v], upairs, usem)

    for l in range(L):
      @plsc.parallel_loop(0, BPW // LANES)
      def _(j0, l=l):
        v = midx_v[l, pl.ds(j0 * LANES, LANES)]
        mrow_v[l, pl.ds(j0 * LANES, LANES)] = v >> 1
        mcol_v[l, pl.ds(j0 * LANES, LANES)] = (v & 1) << 6

    copies = [None] * L
    for l in range(NBUF):
      copies[l] = pltpu.async_copy(
          mt_hbm.at[mrow_v.at[l]], mring.at[l % NBUF], msems[l % NBUF])

    zeros = jnp.zeros((LANES,), jnp.float32)

    @plsc.parallel_loop(0, D)
    def _(c):
      for j0 in range(BPW // LANES):
        pooled[c, pl.ds(j0 * LANES, LANES)] = zeros

    lanes = lax.iota(jnp.int32, LANES)
    for l in range(L):
      copies[l].wait()
      buf = mring.at[l % NBUF]

      @plsc.parallel_loop(0, BPW // LANES)
      def _(j0, buf=buf, l=l):
        jvec = lanes + j0 * LANES
        par = mcol_v[l, pl.ds(j0 * LANES, LANES)]

        @plsc.parallel_loop(0, D, unroll=8)
        def _(c, buf=buf, jvec=jvec, par=par, j0=j0):
          val = plsc.load_gather(buf, [jvec, par + c])
          plsc.addupdate(pooled.at[c, pl.ds(j0 * LANES, LANES)], val)

      if l + NBUF < L:
        copies[l + NBUF] = pltpu.async_copy(
            mt_hbm.at[mrow_v.at[l + NBUF]], mring.at[l % NBUF],
            msems[l % NBUF])

    pltpu.sync_copy(pooled, me_out.at[:, pl.ds(base, BPW)])

    ucopy.wait()

    @plsc.parallel_loop(0, BPW // LANES)
    def _(j0):
      jvec = lanes + j0 * LANES
      par = ucol_v[pl.ds(j0 * LANES, LANES)]

      @plsc.parallel_loop(0, D, unroll=8)
      def _(c, jvec=jvec, par=par, j0=j0):
        usel[c, pl.ds(j0 * LANES, LANES)] = plsc.load_gather(
            upairs, [jvec, par + c])

    pltpu.sync_copy(usel, ue_out.at[:, pl.ds(base, BPW)])

  return k(user_indices, midx_t, ut2, mt2)


def _mlp(ue_t, me_t, W1, b1, W2, b2, W3, b3):
  BM = 512
  dn = (((0,), (0,)), ((), ()))

  def body(ue_ref, me_ref, w1_ref, b1_ref, w2_ref, b2_ref, w3_ref, b3_ref,
           o_ref):
    h = lax.dot_general(ue_ref[...], w1_ref[:D], dn,
                        preferred_element_type=jnp.float32)
    h = h + lax.dot_general(me_ref[...], w1_ref[D:], dn,
                            preferred_element_type=jnp.float32)
    h = jax.nn.relu(h + b1_ref[...])
    h = jax.nn.relu(
        jnp.dot(h, w2_ref[...], preferred_element_type=jnp.float32)
        + b2_ref[...])
    o_ref[...] = (jnp.dot(h, w3_ref[...], preferred_element_type=jnp.float32)
                  + b3_ref[...])

  grid = (B // BM,)
  return pl.pallas_call(
      body,
      grid=grid,
      in_specs=[
          pl.BlockSpec((D, BM), lambda i: (0, i)),
          pl.BlockSpec((D, BM), lambda i: (0, i)),
          pl.BlockSpec((2 * D, 256), lambda i: (0, 0)),
          pl.BlockSpec((1, 256), lambda i: (0, 0)),
          pl.BlockSpec((256, 128), lambda i: (0, 0)),
          pl.BlockSpec((1, 128), lambda i: (0, 0)),
          pl.BlockSpec((128, 1), lambda i: (0, 0)),
          pl.BlockSpec((1, 1), lambda i: (0, 0)),
      ],
      out_specs=pl.BlockSpec((BM, 1), lambda i: (i, 0)),
      out_shape=jax.ShapeDtypeStruct((B, 1), jnp.float32),
  )(ue_t, me_t, W1, b1.reshape(1, 256), W2, b2.reshape(1, 128), W3,
    b3.reshape(1, 1))


def kernel(user_indices, movie_indices, user_table, movie_table,
           W1, b1, W2, b2, W3, b3):
  midx_t = movie_indices.astype(jnp.int32).T
  ut2 = user_table.reshape(V2, PW)
  mt2 = movie_table.reshape(V2, PW)
  ue_t, me_t = _sc_pooled_lookup(
      user_indices.astype(jnp.int32), midx_t, ut2, mt2)
  pred = _mlp(ue_t, me_t, W1, b1, W2, b2, W3, b3)
  return pred.squeeze(-1)
